# baseline (speedup 1.0000x reference)
import jax
import jax.numpy as jnp
from jax.experimental import pallas as pl
from jax.experimental.pallas import tpu as pltpu

N, E, A, TK = 16384, 64, 64, 32
R, C = 128, 128
NCH = 4
CH = N // NCH
RCH = CH // C


def _body(g_hbm, s_hbm, wq_ref, bq_ref, wk_ref, bk_ref, wv_ref, bv_ref,
          wp1_ref, bp1_ref, wp2_ref, bp2_ref, wf1_ref, bf1_ref, wf2_ref,
          bf2_ref, out_hbm, aw_ref, idx_ref, tw_ref,
          g_v, s_v, o_v, sc_v, *sems):
    g_sems = sems[:NCH]
    s_sems = sems[NCH:2 * NCH]
    o_sems = sems[2 * NCH:]

    g_cps, s_cps = [], []
    for i in range(NCH):
        cp = pltpu.make_async_copy(g_hbm.at[pl.ds(i * CH, CH), :],
                                   g_v.at[pl.ds(i * CH, CH), :], g_sems[i])
        cp.start(); g_cps.append(cp)
        cp = pltpu.make_async_copy(s_hbm.at[pl.ds(i * CH, CH), :],
                                   s_v.at[pl.ds(i * CH, CH), :], s_sems[i])
        cp.start(); s_cps.append(cp)

    for i in range(NCH):
        g_cps[i].wait()
        s_cps[i].wait()
        gi = g_v[pl.ds(i * CH, CH), :]
        si = s_v[pl.ds(i * CH, CH), :]
        q = jnp.dot(si, wq_ref[...], preferred_element_type=jnp.float32) \
            + bq_ref[...]
        k = jnp.dot(gi, wk_ref[...], preferred_element_type=jnp.float32) \
            + bk_ref[...]
        score = jnp.sum(q * k, axis=1) * (1.0 / (A ** 0.5))
        sc_v[pl.ds(i * RCH, RCH), :] = score.reshape(RCH, C)

    sc2d = sc_v[...]
    mx = jnp.max(sc2d)
    ex = jnp.exp(sc2d - mx)
    aw = ex / jnp.sum(ex)
    aw_ref[...] = aw

    BIG = jnp.int32(1 << 30)
    row_i = jax.lax.broadcasted_iota(jnp.int32, (R, C), 0)
    col_i = jax.lax.broadcasted_iota(jnp.int32, (R, C), 1)
    flat_i = row_i * C + col_i
    lane = jax.lax.broadcasted_iota(jnp.int32, (1, C), 1)
    lane32 = jax.lax.broadcasted_iota(jnp.int32, (1, TK), 1)

    NQ = 4
    QR = R // NQ
    flat_q = [flat_i[q * QR:(q + 1) * QR, :] for q in range(NQ)]
    quads = [aw[q * QR:(q + 1) * QR, :] for q in range(NQ)]

    def step(i, carry):
        a, idx_acc, w_acc = carry
        a2, i2, w2 = [], [], []
        for q in range(NQ):
            m = jnp.max(a[q], keepdims=True)
            mb = jnp.broadcast_to(m, (QR, C))
            key = jnp.where(a[q] == mb, flat_q[q], BIG)
            fidx = jnp.min(key, keepdims=True)
            fb = jnp.broadcast_to(fidx, (QR, C))
            a2.append(jnp.where(flat_q[q] == fb, jnp.float32(-1.0), a[q]))
            i2.append(jnp.where(lane32 == i,
                                jnp.broadcast_to(fidx, (1, TK)), idx_acc[q]))
            w2.append(jnp.where(lane32 == i,
                                jnp.broadcast_to(m, (1, TK)), w_acc[q]))
        return tuple(a2), tuple(i2), tuple(w2)

    init = (tuple(quads),
            tuple(jnp.zeros((1, TK), jnp.int32) for _ in range(NQ)),
            tuple(jnp.zeros((1, TK), jnp.float32) for _ in range(NQ)))
    _, idx_q, w_q = jax.lax.fori_loop(0, TK, step, init)

    candv = jnp.concatenate(list(w_q), axis=1)
    candi = jnp.concatenate(list(idx_q), axis=1)

    def mstep(i, carry):
        cv, idx_acc, w_acc = carry
        m = jnp.max(cv, keepdims=True)
        mb = jnp.broadcast_to(m, (1, C))
        key = jnp.where(cv == mb, candi, BIG)
        fidx = jnp.min(key, keepdims=True)
        fb = jnp.broadcast_to(fidx, (1, C))
        cv = jnp.where(candi == fb, jnp.float32(-1.0), cv)
        idx_acc = jnp.where(lane32 == i, jnp.broadcast_to(fidx, (1, TK)),
                            idx_acc)
        w_acc = jnp.where(lane32 == i, jnp.broadcast_to(m, (1, TK)), w_acc)
        return cv, idx_acc, w_acc

    minit = (candv, jnp.zeros((1, TK), jnp.int32),
             jnp.zeros((1, TK), jnp.float32))
    _, idx_acc, w_acc = jax.lax.fori_loop(0, TK, mstep, minit)
    idx_ref[...] = idx_acc
    tw_ref[...] = w_acc

    ws = jnp.zeros((1, E), jnp.float32)
    for i in range(TK):
        sel = lane32 == i
        fi = jnp.max(jnp.where(sel, idx_acc, 0))
        wi = jnp.max(jnp.where(sel, w_acc, jnp.float32(0.0)))
        ws = ws + wi * s_v[pl.ds(fi, 1), :]

    wsum = jnp.sum(w_acc)
    pc0 = jnp.dot(ws, wv_ref[...], preferred_element_type=jnp.float32) \
        + wsum * bv_ref[...]
    h = jnp.maximum(
        jnp.dot(pc0, wp1_ref[...], preferred_element_type=jnp.float32)
        + bp1_ref[...], 0.0)
    pc = jnp.dot(h, wp2_ref[...], preferred_element_type=jnp.float32) \
        + bp2_ref[...]
    c_row = jnp.dot(pc, wf1_ref[E:, :], preferred_element_type=jnp.float32) \
        + bf1_ref[...]

    o_cps = []
    for i in range(NCH):
        gi = g_v[pl.ds(i * CH, CH), :]
        h2 = jnp.maximum(
            jnp.dot(gi, wf1_ref[:E, :], preferred_element_type=jnp.float32)
            + c_row, 0.0)
        o_v[pl.ds(i * CH, CH), :] = \
            jnp.dot(h2, wf2_ref[...], preferred_element_type=jnp.float32) \
            + bf2_ref[...]
        cp = pltpu.make_async_copy(o_v.at[pl.ds(i * CH, CH), :],
                                   out_hbm.at[pl.ds(i * CH, CH), :],
                                   o_sems[i])
        cp.start(); o_cps.append(cp)
    for cp in o_cps:
        cp.wait()


def kernel(graph_repr, substructure_repr, Wq, bq, Wk, bk, Wv, bv,
           Wp1, bp1, Wp2, bp2, Wf1, bf1, Wf2, bf2):
    out, aw, idx, tw = pl.pallas_call(
        _body,
        in_specs=[pl.BlockSpec(memory_space=pl.ANY),
                  pl.BlockSpec(memory_space=pl.ANY)]
        + [pl.BlockSpec(x.shape, lambda: (0, 0))
           for x in (Wq, bq.reshape(1, A), Wk, bk.reshape(1, A),
                     Wv, bv.reshape(1, E), Wp1, bp1.reshape(1, A),
                     Wp2, bp2.reshape(1, E), Wf1, bf1.reshape(1, A),
                     Wf2, bf2.reshape(1, E))],
        out_specs=[
            pl.BlockSpec(memory_space=pl.ANY),
            pl.BlockSpec((R, C), lambda: (0, 0)),
            pl.BlockSpec((1, TK), lambda: (0, 0)),
            pl.BlockSpec((1, TK), lambda: (0, 0)),
        ],
        out_shape=[
            jax.ShapeDtypeStruct((N, E), jnp.float32),
            jax.ShapeDtypeStruct((R, C), jnp.float32),
            jax.ShapeDtypeStruct((1, TK), jnp.int32),
            jax.ShapeDtypeStruct((1, TK), jnp.float32),
        ],
        scratch_shapes=[
            pltpu.VMEM((N, E), jnp.float32),
            pltpu.VMEM((N, E), jnp.float32),
            pltpu.VMEM((N, E), jnp.float32),
            pltpu.VMEM((R, C), jnp.float32),
        ] + [pltpu.SemaphoreType.DMA] * (3 * NCH),
    )(graph_repr, substructure_repr,
      Wq, bq.reshape(1, A), Wk, bk.reshape(1, A), Wv, bv.reshape(1, E),
      Wp1, bp1.reshape(1, A), Wp2, bp2.reshape(1, E),
      Wf1, bf1.reshape(1, A), Wf2, bf2.reshape(1, E))
    return out, aw.reshape(N), idx.reshape(TK), tw.reshape(TK)


# speedup vs baseline: 1.6485x; 1.6485x over previous
import jax
import jax.numpy as jnp
from jax.experimental import pallas as pl
from jax.experimental.pallas import tpu as pltpu

N, E, A, TK = 16384, 64, 64, 32
R, C = 128, 128
NCH = 4
CH = N // NCH
RCH = CH // C


def _body(g_hbm, s_hbm, wq_ref, bq_ref, wk_ref, bk_ref, wv_ref, bv_ref,
          wp1_ref, bp1_ref, wp2_ref, bp2_ref, wf1_ref, bf1_ref, wf2_ref,
          bf2_ref, out_hbm, aw_ref, idx_ref, tw_ref,
          g_v, s_v, o_v, sc_v, *sems):
    g_sems = sems[:NCH]
    s_sems = sems[NCH:2 * NCH]
    o_sems = sems[2 * NCH:]

    g_cps, s_cps = [], []
    for i in range(NCH):
        cp = pltpu.make_async_copy(g_hbm.at[pl.ds(i * CH, CH), :],
                                   g_v.at[pl.ds(i * CH, CH), :], g_sems[i])
        cp.start(); g_cps.append(cp)
        cp = pltpu.make_async_copy(s_hbm.at[pl.ds(i * CH, CH), :],
                                   s_v.at[pl.ds(i * CH, CH), :], s_sems[i])
        cp.start(); s_cps.append(cp)

    for i in range(NCH):
        g_cps[i].wait()
        s_cps[i].wait()
        gi = g_v[pl.ds(i * CH, CH), :]
        si = s_v[pl.ds(i * CH, CH), :]
        q = jnp.dot(si, wq_ref[...], preferred_element_type=jnp.float32) \
            + bq_ref[...]
        k = jnp.dot(gi, wk_ref[...], preferred_element_type=jnp.float32) \
            + bk_ref[...]
        score = jnp.sum(q * k, axis=1) * (1.0 / (A ** 0.5))
        sc_v[pl.ds(i * RCH, RCH), :] = score.reshape(RCH, C)

    sc2d = sc_v[...]
    mx = jnp.max(sc2d)
    ex = jnp.exp(sc2d - mx)
    aw = ex / jnp.sum(ex)
    aw_ref[...] = aw

    BIG = jnp.int32(1 << 30)
    row_i = jax.lax.broadcasted_iota(jnp.int32, (R, C), 0)
    col_i = jax.lax.broadcasted_iota(jnp.int32, (R, C), 1)
    flat_i = row_i * C + col_i
    lane = jax.lax.broadcasted_iota(jnp.int32, (1, C), 1)
    lane32 = jax.lax.broadcasted_iota(jnp.int32, (1, TK), 1)

    def step(i, carry):
        a, idx_acc, w_acc = carry
        colmax = jnp.max(a, axis=0, keepdims=True)
        cmb = jnp.broadcast_to(colmax, (R, C))
        colrow = jnp.min(jnp.where(a == cmb, row_i, BIG), axis=0,
                         keepdims=True)
        m = jnp.max(colmax, keepdims=True)
        mb = jnp.broadcast_to(m, (1, C))
        key = jnp.where(colmax == mb, colrow * C + lane, BIG)
        fidx = jnp.min(key, keepdims=True)
        fb = jnp.broadcast_to(fidx, (R, C))
        a = jnp.where(flat_i == fb, jnp.float32(-1.0), a)
        idx_acc = jnp.where(lane32 == i, jnp.broadcast_to(fidx, (1, TK)),
                            idx_acc)
        w_acc = jnp.where(lane32 == i, jnp.broadcast_to(m, (1, TK)), w_acc)
        return a, idx_acc, w_acc

    init = (aw, jnp.zeros((1, TK), jnp.int32), jnp.zeros((1, TK), jnp.float32))
    _, idx_acc, w_acc = jax.lax.fori_loop(0, TK, step, init)
    idx_ref[...] = idx_acc
    tw_ref[...] = w_acc

    ws = jnp.zeros((1, E), jnp.float32)
    for i in range(TK):
        sel = lane32 == i
        fi = jnp.max(jnp.where(sel, idx_acc, 0))
        wi = jnp.max(jnp.where(sel, w_acc, jnp.float32(0.0)))
        ws = ws + wi * s_v[pl.ds(fi, 1), :]

    wsum = jnp.sum(w_acc)
    pc0 = jnp.dot(ws, wv_ref[...], preferred_element_type=jnp.float32) \
        + wsum * bv_ref[...]
    h = jnp.maximum(
        jnp.dot(pc0, wp1_ref[...], preferred_element_type=jnp.float32)
        + bp1_ref[...], 0.0)
    pc = jnp.dot(h, wp2_ref[...], preferred_element_type=jnp.float32) \
        + bp2_ref[...]
    c_row = jnp.dot(pc, wf1_ref[E:, :], preferred_element_type=jnp.float32) \
        + bf1_ref[...]

    o_cps = []
    for i in range(NCH):
        gi = g_v[pl.ds(i * CH, CH), :]
        h2 = jnp.maximum(
            jnp.dot(gi, wf1_ref[:E, :], preferred_element_type=jnp.float32)
            + c_row, 0.0)
        o_v[pl.ds(i * CH, CH), :] = \
            jnp.dot(h2, wf2_ref[...], preferred_element_type=jnp.float32) \
            + bf2_ref[...]
        cp = pltpu.make_async_copy(o_v.at[pl.ds(i * CH, CH), :],
                                   out_hbm.at[pl.ds(i * CH, CH), :],
                                   o_sems[i])
        cp.start(); o_cps.append(cp)
    for cp in o_cps:
        cp.wait()


def kernel(graph_repr, substructure_repr, Wq, bq, Wk, bk, Wv, bv,
           Wp1, bp1, Wp2, bp2, Wf1, bf1, Wf2, bf2):
    out, aw, idx, tw = pl.pallas_call(
        _body,
        in_specs=[pl.BlockSpec(memory_space=pl.ANY),
                  pl.BlockSpec(memory_space=pl.ANY)]
        + [pl.BlockSpec(x.shape, lambda: (0, 0))
           for x in (Wq, bq.reshape(1, A), Wk, bk.reshape(1, A),
                     Wv, bv.reshape(1, E), Wp1, bp1.reshape(1, A),
                     Wp2, bp2.reshape(1, E), Wf1, bf1.reshape(1, A),
                     Wf2, bf2.reshape(1, E))],
        out_specs=[
            pl.BlockSpec(memory_space=pl.ANY),
            pl.BlockSpec((R, C), lambda: (0, 0)),
            pl.BlockSpec((1, TK), lambda: (0, 0)),
            pl.BlockSpec((1, TK), lambda: (0, 0)),
        ],
        out_shape=[
            jax.ShapeDtypeStruct((N, E), jnp.float32),
            jax.ShapeDtypeStruct((R, C), jnp.float32),
            jax.ShapeDtypeStruct((1, TK), jnp.int32),
            jax.ShapeDtypeStruct((1, TK), jnp.float32),
        ],
        scratch_shapes=[
            pltpu.VMEM((N, E), jnp.float32),
            pltpu.VMEM((N, E), jnp.float32),
            pltpu.VMEM((N, E), jnp.float32),
            pltpu.VMEM((R, C), jnp.float32),
        ] + [pltpu.SemaphoreType.DMA] * (3 * NCH),
    )(graph_repr, substructure_repr,
      Wq, bq.reshape(1, A), Wk, bk.reshape(1, A), Wv, bv.reshape(1, E),
      Wp1, bp1.reshape(1, A), Wp2, bp2.reshape(1, E),
      Wf1, bf1.reshape(1, A), Wf2, bf2.reshape(1, E))
    return out, aw.reshape(N), idx.reshape(TK), tw.reshape(TK)
